# baseline (speedup 1.0000x reference)
import functools

import jax
import jax.numpy as jnp
from jax import lax
from jax.experimental import pallas as pl
from jax.experimental.pallas import tpu as pltpu
from jax.experimental.pallas import tpu_sc as plsc

BATCH = 16384
EMBED = 64
LANES = 16
NUM_CORES = 2
NUM_SUBCORES = 16
NUM_WORKERS = NUM_CORES * NUM_SUBCORES
BPW = BATCH // NUM_WORKERS
CHUNK = 128
NCHUNK = BPW // CHUNK


@functools.partial(
    pl.kernel,
    out_type=jax.ShapeDtypeStruct((BATCH,), jnp.float32),
    mesh=plsc.VectorSubcoreMesh(
        core_axis_name="c", subcore_axis_name="s",
        num_cores=NUM_CORES, num_subcores=NUM_SUBCORES),
    scratch_types=[
        pltpu.VMEM((NCHUNK, CHUNK), jnp.int32),
        pltpu.VMEM((NCHUNK, CHUNK), jnp.int32),
        pltpu.VMEM((BPW, EMBED), jnp.float32),
        pltpu.VMEM((BPW, EMBED), jnp.float32),
        pltpu.VMEM((BPW, LANES + 1), jnp.float32),
        pltpu.VMEM((BPW,), jnp.float32),
        pltpu.SemaphoreType.DMA,
        pltpu.SemaphoreType.DMA,
    ],
    compiler_params=pltpu.CompilerParams(
        needs_layout_passes=False, use_tc_tiling_on_sc=False),
)
def _sc_kernel(uidx_hbm, midx_hbm, ut_hbm, mt_hbm, out_hbm,
               uidx_v, midx_v, urows_v, mrows_v, part_v, out_v, usem, msem):
    wid = lax.axis_index("s") * NUM_CORES + lax.axis_index("c")
    base = wid * BPW

    pltpu.sync_copy(uidx_hbm.at[pl.ds(wid * NCHUNK, NCHUNK)], uidx_v)
    pltpu.sync_copy(midx_hbm.at[pl.ds(wid * NCHUNK, NCHUNK)], midx_v)

    copies = []
    for j in range(NCHUNK):
        copies.append(pltpu.async_copy(
            ut_hbm.at[uidx_v.at[j]],
            urows_v.at[pl.ds(j * CHUNK, CHUNK)], usem))
        copies.append(pltpu.async_copy(
            mt_hbm.at[midx_v.at[j]],
            mrows_v.at[pl.ds(j * CHUNK, CHUNK)], msem))
    for c in copies:
        c.wait()

    def body(i, _):
        acc = urows_v[i, pl.ds(0, LANES)] * mrows_v[i, pl.ds(0, LANES)]
        for k in range(1, EMBED // LANES):
            acc = acc + (urows_v[i, pl.ds(k * LANES, LANES)]
                         * mrows_v[i, pl.ds(k * LANES, LANES)])
        part_v[i, pl.ds(0, LANES)] = acc
        return 0

    lax.fori_loop(0, BPW, body, 0, unroll=4)

    def body2(g, _):
        rows = g * LANES + lax.iota(jnp.int32, LANES)
        acc = plsc.load_gather(part_v, [rows, jnp.zeros((LANES,), jnp.int32)])
        for l in range(1, LANES):
            acc = acc + plsc.load_gather(
                part_v, [rows, jnp.full((LANES,), l, jnp.int32)])
        out_v[pl.ds(g * LANES, LANES)] = acc
        return 0

    lax.fori_loop(0, BPW // LANES, body2, 0, unroll=2)

    pltpu.sync_copy(out_v, out_hbm.at[pl.ds(base, BPW)])


def kernel(userIndices, movieIndices, user_table, movie_table):
    uidx = userIndices.astype(jnp.int32).reshape(NUM_WORKERS * NCHUNK, CHUNK)
    midx = movieIndices.astype(jnp.int32).reshape(NUM_WORKERS * NCHUNK, CHUNK)
    return _sc_kernel(uidx, midx, user_table, movie_table)


# speedup vs baseline: 2.1199x; 2.1199x over previous
import functools

import jax
import jax.numpy as jnp
from jax import lax
from jax.experimental import pallas as pl
from jax.experimental.pallas import tpu as pltpu
from jax.experimental.pallas import tpu_sc as plsc

BATCH = 16384
EMBED = 64
LANES = 16
SUBL = 8
NUM_CORES = 2
NUM_SUBCORES = 16
NUM_WORKERS = NUM_CORES * NUM_SUBCORES
BPW = BATCH // NUM_WORKERS
CH = 16
NCH = BPW // CH
NSEG = EMBED // LANES


@functools.partial(
    pl.kernel,
    out_type=jax.ShapeDtypeStruct((BATCH,), jnp.float32),
    mesh=plsc.VectorSubcoreMesh(
        core_axis_name="c", subcore_axis_name="s",
        num_cores=NUM_CORES, num_subcores=NUM_SUBCORES),
    scratch_types=[
        pltpu.VMEM((BPW,), jnp.int32),
        pltpu.VMEM((BPW,), jnp.int32),
        pltpu.VMEM((CH, SUBL, EMBED), jnp.float32),
        pltpu.VMEM((CH, SUBL, EMBED), jnp.float32),
        pltpu.VMEM((CH, SUBL, EMBED), jnp.float32),
        pltpu.VMEM((CH, SUBL, EMBED), jnp.float32),
        pltpu.VMEM((CH, LANES + 1), jnp.float32),
        pltpu.VMEM((BPW,), jnp.float32),
        pltpu.SemaphoreType.DMA,
        pltpu.SemaphoreType.DMA,
        pltpu.SemaphoreType.DMA,
        pltpu.SemaphoreType.DMA,
    ],
    compiler_params=pltpu.CompilerParams(
        needs_layout_passes=False, use_tc_tiling_on_sc=True),
)
def _sc_kernel(uidx_hbm, midx_hbm, ut_hbm, mt_hbm, out_hbm,
               uidx_v, midx_v, u0, u1, m0, m1, part_v, out_v,
               usem0, usem1, msem0, msem1):
    wid = lax.axis_index("s") * NUM_CORES + lax.axis_index("c")
    base = wid * BPW

    pltpu.sync_copy(uidx_hbm.at[pl.ds(base, BPW)], uidx_v)
    pltpu.sync_copy(midx_hbm.at[pl.ds(base, BPW)], midx_v)

    def fire(c, ubuf, mbuf, usem, msem):
        ublk = lax.shift_right_logical(uidx_v[pl.ds(c * CH, LANES)], 3)
        mblk = lax.shift_right_logical(midx_v[pl.ds(c * CH, LANES)], 3)
        for j in range(CH):
            pltpu.async_copy(ut_hbm.at[ublk[j]], ubuf.at[j], usem)
            pltpu.async_copy(mt_hbm.at[mblk[j]], mbuf.at[j], msem)

    def wait(ubuf, mbuf, usem, msem):
        pltpu.make_async_copy(ut_hbm.at[pl.ds(0, CH)], ubuf, usem).wait()
        pltpu.make_async_copy(mt_hbm.at[pl.ds(0, CH)], mbuf, msem).wait()

    def compute(c, ubuf, mbuf):
        ur8 = jnp.bitwise_and(uidx_v[pl.ds(c * CH, LANES)], 7)
        mr8 = jnp.bitwise_and(midx_v[pl.ds(c * CH, LANES)], 7)
        for j in range(CH):
            ru = ur8[j]
            rm = mr8[j]
            acc = (ubuf[j, ru, pl.ds(0, LANES)]
                   * mbuf[j, rm, pl.ds(0, LANES)])
            for k in range(1, NSEG):
                acc = acc + (ubuf[j, ru, pl.ds(k * LANES, LANES)]
                             * mbuf[j, rm, pl.ds(k * LANES, LANES)])
            part_v[j, pl.ds(0, LANES)] = acc
        rows = lax.iota(jnp.int32, LANES)
        acc = plsc.load_gather(part_v, [rows, jnp.zeros((LANES,), jnp.int32)])
        for l in range(1, LANES):
            acc = acc + plsc.load_gather(
                part_v, [rows, jnp.full((LANES,), l, jnp.int32)])
        out_v[pl.ds(c * CH, CH)] = acc

    fire(0, u0, m0, usem0, msem0)

    def pair(t, _):
        ca = 2 * t
        fire(ca + 1, u1, m1, usem1, msem1)
        wait(u0, m0, usem0, msem0)
        compute(ca, u0, m0)

        @pl.when(t < NCH // 2 - 1)
        def _():
            fire(ca + 2, u0, m0, usem0, msem0)

        wait(u1, m1, usem1, msem1)
        compute(ca + 1, u1, m1)
        return 0

    lax.fori_loop(0, NCH // 2, pair, 0)

    pltpu.sync_copy(out_v, out_hbm.at[pl.ds(base, BPW)])


def kernel(userIndices, movieIndices, user_table, movie_table):
    ut3 = user_table.reshape(-1, SUBL, EMBED)
    mt3 = movie_table.reshape(-1, SUBL, EMBED)
    return _sc_kernel(userIndices.astype(jnp.int32),
                      movieIndices.astype(jnp.int32),
                      ut3, mt3)
